# baseline (speedup 1.0000x reference)
import functools

import jax
import jax.numpy as jnp
from jax import lax
from jax.experimental import pallas as pl
from jax.experimental.pallas import tpu as pltpu
from jax.experimental.pallas import tpu_sc as plsc

N = 100000
E = 1600000
NG = 64
NP = 102400
BLK = 4096
GRID = NP // BLK

NW = 32
BE = 1584
NB = 32
EPW = BE * NB
EP = EPW * NW
ZCP = 640
RPT = NP // 16

_SC_PARAMS = pltpu.CompilerParams(use_tc_tiling_on_sc=False)


@functools.lru_cache(maxsize=None)
def _sc_kernels():
    mesh = plsc.VectorSubcoreMesh(core_axis_name="c", subcore_axis_name="s",
                                  num_cores=2, num_subcores=16)

    def _edge_body(n_chunks):
        def body(*refs):
            ei_hbm = refs[0]
            hs_list = refs[1:1 + n_chunks]
            out_list = refs[1 + n_chunks:1 + 2 * n_chunks]
            idx0, rows0, acc_sh, semi0, semr0 = refs[1 + 2 * n_chunks:]
            c = lax.axis_index("c")
            s = lax.axis_index("s")
            wid = c * 16 + s
            tb = wid * EPW

            for hs_hbm, out_hbm in zip(hs_list, out_list):
                @pl.loop(0, BE)
                def _(j):
                    rows0[j] = jnp.zeros((16,), jnp.float32)

                @pl.loop(0, RPT // ZCP)
                def _(j):
                    pltpu.sync_copy(rows0.at[pl.ds(0, ZCP)],
                                    acc_sh.at[pl.ds(s * RPT + j * ZCP, ZCP)])

                plsc.subcore_barrier()

                @pl.loop(0, NB)
                def _(b):
                    pltpu.sync_copy(ei_hbm.at[:, pl.ds(tb + b * BE, BE)],
                                    idx0)
                    pltpu.async_copy(hs_hbm.at[idx0.at[0]], rows0,
                                     semr0).wait()
                    pltpu.sync_copy(rows0, acc_sh.at[idx0.at[1]], add=True)

                plsc.subcore_barrier()

                @pl.loop(0, RPT // ZCP)
                def _(j):
                    r0 = s * RPT + j * ZCP
                    pltpu.sync_copy(acc_sh.at[pl.ds(r0, ZCP)],
                                    out_hbm.at[c, pl.ds(r0, ZCP)])
        return body

    def _make_edge_pass(n_chunks):
        return pl.kernel(
            _edge_body(n_chunks), mesh=mesh,
            out_type=(jax.ShapeDtypeStruct((2, NP, 16), jnp.float32)
                      if n_chunks == 1 else
                      tuple(jax.ShapeDtypeStruct((2, NP, 16), jnp.float32)
                            for _ in range(n_chunks))),
            compiler_params=_SC_PARAMS,
            scratch_types=[
                pltpu.VMEM((2, BE), jnp.int32),
                pltpu.VMEM((BE, 16), jnp.float32),
                pltpu.VMEM_SHARED((NP, 16), jnp.float32),
                pltpu.SemaphoreType.DMA,
                pltpu.SemaphoreType.DMA,
            ])

    _sc_edge_pass1 = _make_edge_pass(1)
    _sc_edge_pass2 = _make_edge_pass(2)

    @functools.partial(
        pl.kernel, mesh=mesh,
        out_type=jax.ShapeDtypeStruct((2, NP, 16), jnp.float32),
        compiler_params=_SC_PARAMS,
        scratch_types=[
            pltpu.VMEM((BE,), jnp.int32),
            pltpu.VMEM((BE, 16), jnp.float32),
            pltpu.VMEM_SHARED((NP, 16), jnp.float32),
            pltpu.SemaphoreType.DMA,
        ])
    def _sc_deg_pass(ei_hbm, out_hbm, dst_v, ones_v, acc_sh, sem):
        c = lax.axis_index("c")
        s = lax.axis_index("s")
        wid = c * 16 + s

        @pl.loop(0, BE)
        def _(j):
            ones_v[j] = jnp.zeros((16,), jnp.float32)

        @pl.loop(0, RPT // ZCP)
        def _(j):
            pltpu.sync_copy(ones_v.at[pl.ds(0, ZCP)],
                            acc_sh.at[pl.ds(s * RPT + j * ZCP, ZCP)])

        @pl.loop(0, BE)
        def _(j):
            ones_v[j] = jnp.ones((16,), jnp.float32)

        plsc.subcore_barrier()

        @pl.loop(0, NB)
        def _(b):
            base = wid * EPW + b * BE
            pltpu.sync_copy(ei_hbm.at[1, pl.ds(base, BE)], dst_v)
            pltpu.sync_copy(ones_v, acc_sh.at[dst_v], add=True)

        plsc.subcore_barrier()

        @pl.loop(0, RPT // ZCP)
        def _(j):
            r0 = s * RPT + j * ZCP
            pltpu.sync_copy(acc_sh.at[pl.ds(r0, ZCP)],
                            out_hbm.at[c, pl.ds(r0, ZCP)])

    return _sc_edge_pass1, _sc_edge_pass2, _sc_deg_pass


PBLK = BLK // 8


def _tc1_body(x_ref, w_ref, dg_ref, disb_ref, ha_ref, hb_ref):
    dg = dg_ref[...]
    disb = lax.rsqrt(dg[0] + dg[1] + 1.0)
    p = jnp.dot(x_ref[...], w_ref[...], preferred_element_type=jnp.float32)
    disb_ref[...] = disb
    ha_ref[...] = p[:, :128] * disb
    hb_ref[...] = p[:, 128:] * disb


def _tc2_body(aa_ref, ab_ref, ha_ref, hb_ref, disb_ref, wa_ref, wb_ref,
              ba_ref, bb_ref, out_ref):
    disb = disb_ref[...]
    aa = aa_ref[...]
    ab = ab_ref[...]
    ha = jnp.maximum((aa[0] + aa[1] + ha_ref[...]) * disb + ba_ref[...], 0.0)
    hb = jnp.maximum((ab[0] + ab[1] + hb_ref[...]) * disb + bb_ref[...], 0.0)
    p = (jnp.dot(ha, wa_ref[...], preferred_element_type=jnp.float32)
         + jnp.dot(hb, wb_ref[...], preferred_element_type=jnp.float32))
    out_ref[...] = p * disb


def _tc3_body(a_ref, hs_ref, disb_ref, w_ref, b_ref, out_ref):
    disb = disb_ref[...]
    a = a_ref[...]
    h = jnp.maximum((a[0] + a[1] + hs_ref[...]) * disb + b_ref[...], 0.0)
    p = jnp.dot(h, w_ref[...], preferred_element_type=jnp.float32)
    out_ref[...] = p * disb


def _tc4_body(a_ref, hs_ref, disb_ref, b3_ref, bt_ref, we_ref, be_ref,
              wc_ref, bc_ref, emb_ref, out_ref, pooled):
    i = pl.program_id(0)
    a = a_ref[...]
    s = (a[0] + a[1] + hs_ref[...]) * disb_ref[...] + b3_ref[...]
    h = jnp.maximum(s, 0.0)
    bt = bt_ref[...]
    contrib = jnp.zeros((NG, 16), jnp.float32)
    for k in range(8):
        oh = (lax.broadcasted_iota(jnp.int32, (NG, PBLK), 0)
              == bt[k][None, :]).astype(jnp.float32)
        contrib += jnp.dot(oh, h[:, 16 * k:16 * (k + 1)],
                           preferred_element_type=jnp.float32)

    @pl.when(i == 0)
    def _():
        pooled[...] = jnp.zeros_like(pooled)

    pooled[...] += contrib

    @pl.when(i == GRID - 1)
    def _():
        emb = jnp.dot(pooled[...], we_ref[...],
                      preferred_element_type=jnp.float32) + be_ref[...][None, :]
        emb_ref[...] = emb
        out_ref[...] = (jnp.dot(jnp.maximum(emb, 0.0), wc_ref[...],
                                preferred_element_type=jnp.float32)
                        + bc_ref[...][None, :])


def _pk_spec(width=128):
    return pl.BlockSpec((PBLK, width), lambda i: (i, 0))


def _acc_spec():
    return pl.BlockSpec((2, PBLK, 128), lambda i: (0, i, 0))


def _full_spec(shape):
    nd = len(shape)
    return pl.BlockSpec(shape, lambda i: (0,) * nd)


def kernel(x, edge_index, batch, W1, b1, W2, b2, W3, b3, We, be, Wc, bc):
    _sc_edge_pass1, _sc_edge_pass2, _sc_deg_pass = _sc_kernels()
    ei_pad = jnp.concatenate(
        [jnp.zeros((1, EP - E), jnp.int32),
         jnp.full((1, EP - E), N, jnp.int32)], axis=0)
    ei_p = jnp.concatenate([edge_index, ei_pad], axis=1)
    x_p = jnp.pad(x, ((0, NP - N), (0, 0))).reshape(NP // 8, 1024)
    batch_p = jnp.pad(batch, (0, NP - N), constant_values=NG)
    batch_t = batch_p.reshape(NP // 8, 8).T
    eye8 = jnp.eye(8, dtype=jnp.float32)
    w1_big = jnp.concatenate([jnp.kron(eye8, W1[:, :16]),
                              jnp.kron(eye8, W1[:, 16:])], axis=1)
    w2a = jnp.kron(eye8, W2[:16, :])
    w2b = jnp.kron(eye8, W2[16:, :])
    w3_bd = jnp.kron(eye8, W3)
    b1a = jnp.tile(b1[:16], 8)
    b1b = jnp.tile(b1[16:], 8)
    b2t = jnp.tile(b2, 8)
    b3t = jnp.tile(b3, 8)

    def pk(a):
        return a.reshape(a.shape[:-2] + (NP // 8, 128))

    def rows(a):
        return a.reshape(NP, 16)

    dacc = pk(_sc_deg_pass(ei_p))

    disb, hs1a, hs1b = pl.pallas_call(
        _tc1_body,
        grid=(GRID,),
        in_specs=[pl.BlockSpec((PBLK, 1024), lambda i: (i, 0)),
                  _full_spec((1024, 256)), _acc_spec()],
        out_specs=[_pk_spec(), _pk_spec(), _pk_spec()],
        out_shape=[jax.ShapeDtypeStruct((NP // 8, 128), jnp.float32)] * 3,
    )(x_p, w1_big, dacc)

    acc1a, acc1b = _sc_edge_pass2(ei_p, rows(hs1a), rows(hs1b))

    hs2 = pl.pallas_call(
        _tc2_body,
        grid=(GRID,),
        in_specs=[_acc_spec(), _acc_spec(), _pk_spec(), _pk_spec(),
                  _pk_spec(), _full_spec((128, 128)), _full_spec((128, 128)),
                  _full_spec((128,)), _full_spec((128,))],
        out_specs=_pk_spec(),
        out_shape=jax.ShapeDtypeStruct((NP // 8, 128), jnp.float32),
    )(pk(acc1a), pk(acc1b), hs1a, hs1b, disb, w2a, w2b, b1a, b1b)

    acc2 = _sc_edge_pass1(ei_p, rows(hs2))

    hs3 = pl.pallas_call(
        _tc3_body,
        grid=(GRID,),
        in_specs=[_acc_spec(), _pk_spec(), _pk_spec(),
                  _full_spec((128, 128)), _full_spec((128,))],
        out_specs=_pk_spec(),
        out_shape=jax.ShapeDtypeStruct((NP // 8, 128), jnp.float32),
    )(pk(acc2), hs2, disb, w3_bd, b2t)

    acc3 = _sc_edge_pass1(ei_p, rows(hs3))

    embedding, output = pl.pallas_call(
        _tc4_body,
        grid=(GRID,),
        in_specs=[_acc_spec(), _pk_spec(), _pk_spec(),
                  _full_spec((128,)), pl.BlockSpec((8, PBLK), lambda i: (0, i)),
                  _full_spec((16, 16)), _full_spec((16,)),
                  _full_spec((16, 10)), _full_spec((10,))],
        out_specs=[_full_spec((NG, 16)), _full_spec((NG, 10))],
        out_shape=[jax.ShapeDtypeStruct((NG, 16), jnp.float32),
                   jax.ShapeDtypeStruct((NG, 10), jnp.float32)],
        scratch_shapes=[pltpu.VMEM((NG, 16), jnp.float32)],
    )(pk(acc3), hs3, disb, b3t, batch_t, We, be, Wc, bc)

    return (embedding, output)


# speedup vs baseline: 1.3626x; 1.3626x over previous
import functools

import jax
import jax.numpy as jnp
from jax import lax
from jax.experimental import pallas as pl
from jax.experimental.pallas import tpu as pltpu
from jax.experimental.pallas import tpu_sc as plsc

N = 100000
E = 1600000
NG = 64
NP = 102400
BLK = 4096
GRID = NP // BLK

NW = 32
BE = 1000
NB = 50
EPW = BE * NB
ZCP = 640
RPT = NP // 16

_SC_PARAMS = pltpu.CompilerParams(use_tc_tiling_on_sc=False)


@functools.lru_cache(maxsize=None)
def _sc_kernels():
    mesh = plsc.VectorSubcoreMesh(core_axis_name="c", subcore_axis_name="s",
                                  num_cores=2, num_subcores=16)

    def _edge_body(n_chunks):
        def body(*refs):
            ei_hbm = refs[0]
            hs_list = refs[1:1 + n_chunks]
            out_list = refs[1 + n_chunks:1 + 2 * n_chunks]
            src_v, dst_v, rows_v, acc_sh, sem = refs[1 + 2 * n_chunks:]
            c = lax.axis_index("c")
            s = lax.axis_index("s")
            wid = c * 16 + s
            tb = wid * EPW

            for hs_hbm, out_hbm in zip(hs_list, out_list):
                @pl.loop(0, BE)
                def _(j):
                    rows_v[j] = jnp.zeros((16,), jnp.float32)

                @pl.loop(0, RPT // ZCP)
                def _(j):
                    pltpu.sync_copy(rows_v.at[pl.ds(0, ZCP)],
                                    acc_sh.at[pl.ds(s * RPT + j * ZCP, ZCP)])

                plsc.subcore_barrier()

                @pl.loop(0, NB)
                def _(b):
                    base = tb + b * BE
                    pltpu.sync_copy(ei_hbm.at[0, pl.ds(base, BE)], src_v)
                    pltpu.sync_copy(ei_hbm.at[1, pl.ds(base, BE)], dst_v)
                    pltpu.async_copy(hs_hbm.at[src_v], rows_v, sem).wait()
                    pltpu.sync_copy(rows_v, acc_sh.at[dst_v], add=True)

                plsc.subcore_barrier()

                @pl.loop(0, RPT // ZCP)
                def _(j):
                    r0 = s * RPT + j * ZCP
                    pltpu.sync_copy(acc_sh.at[pl.ds(r0, ZCP)],
                                    out_hbm.at[c, pl.ds(r0, ZCP)])
        return body

    def _make_edge_pass(n_chunks):
        return pl.kernel(
            _edge_body(n_chunks), mesh=mesh,
            out_type=(jax.ShapeDtypeStruct((2, NP, 16), jnp.float32)
                      if n_chunks == 1 else
                      tuple(jax.ShapeDtypeStruct((2, NP, 16), jnp.float32)
                            for _ in range(n_chunks))),
            compiler_params=_SC_PARAMS,
            scratch_types=[
                pltpu.VMEM((BE,), jnp.int32),
                pltpu.VMEM((BE,), jnp.int32),
                pltpu.VMEM((BE, 16), jnp.float32),
                pltpu.VMEM_SHARED((NP, 16), jnp.float32),
                pltpu.SemaphoreType.DMA,
            ])

    _sc_edge_pass1 = _make_edge_pass(1)
    _sc_edge_pass2 = _make_edge_pass(2)

    @functools.partial(
        pl.kernel, mesh=mesh,
        out_type=jax.ShapeDtypeStruct((2, NP, 16), jnp.float32),
        compiler_params=_SC_PARAMS,
        scratch_types=[
            pltpu.VMEM((BE,), jnp.int32),
            pltpu.VMEM((BE, 16), jnp.float32),
            pltpu.VMEM_SHARED((NP, 16), jnp.float32),
            pltpu.SemaphoreType.DMA,
        ])
    def _sc_deg_pass(ei_hbm, out_hbm, dst_v, ones_v, acc_sh, sem):
        c = lax.axis_index("c")
        s = lax.axis_index("s")
        wid = c * 16 + s

        @pl.loop(0, BE)
        def _(j):
            ones_v[j] = jnp.zeros((16,), jnp.float32)

        @pl.loop(0, RPT // ZCP)
        def _(j):
            pltpu.sync_copy(ones_v.at[pl.ds(0, ZCP)],
                            acc_sh.at[pl.ds(s * RPT + j * ZCP, ZCP)])

        @pl.loop(0, BE)
        def _(j):
            ones_v[j] = jnp.ones((16,), jnp.float32)

        plsc.subcore_barrier()

        @pl.loop(0, NB)
        def _(b):
            base = wid * EPW + b * BE
            pltpu.sync_copy(ei_hbm.at[1, pl.ds(base, BE)], dst_v)
            pltpu.sync_copy(ones_v, acc_sh.at[dst_v], add=True)

        plsc.subcore_barrier()

        @pl.loop(0, RPT // ZCP)
        def _(j):
            r0 = s * RPT + j * ZCP
            pltpu.sync_copy(acc_sh.at[pl.ds(r0, ZCP)],
                            out_hbm.at[c, pl.ds(r0, ZCP)])

    return _sc_edge_pass1, _sc_edge_pass2, _sc_deg_pass


PBLK = BLK // 8


def _tc1a_body(x_ref, w_ref, p_ref):
    p_ref[...] = jnp.dot(x_ref[...], w_ref[...],
                         preferred_element_type=jnp.float32)


def _tc1b_body(p_ref, dg_ref, disb_ref, ha_ref, hb_ref):
    dg = dg_ref[...]
    disb = lax.rsqrt(dg[0] + dg[1] + 1.0)
    p = p_ref[...]
    disb_ref[...] = disb
    ha_ref[...] = p[:, :128] * disb
    hb_ref[...] = p[:, 128:] * disb


def _tc2_body(aa_ref, ab_ref, ha_ref, hb_ref, disb_ref, wa_ref, wb_ref,
              ba_ref, bb_ref, out_ref):
    disb = disb_ref[...]
    aa = aa_ref[...]
    ab = ab_ref[...]
    ha = jnp.maximum((aa[0] + aa[1] + ha_ref[...]) * disb + ba_ref[...], 0.0)
    hb = jnp.maximum((ab[0] + ab[1] + hb_ref[...]) * disb + bb_ref[...], 0.0)
    p = (jnp.dot(ha, wa_ref[...], preferred_element_type=jnp.float32)
         + jnp.dot(hb, wb_ref[...], preferred_element_type=jnp.float32))
    out_ref[...] = p * disb


def _tc3_body(a_ref, hs_ref, disb_ref, w_ref, b_ref, out_ref):
    disb = disb_ref[...]
    a = a_ref[...]
    h = jnp.maximum((a[0] + a[1] + hs_ref[...]) * disb + b_ref[...], 0.0)
    p = jnp.dot(h, w_ref[...], preferred_element_type=jnp.float32)
    out_ref[...] = p * disb


def _tc4_body(a_ref, hs_ref, disb_ref, b3_ref, bt_ref, we_ref, be_ref,
              wc_ref, bc_ref, emb_ref, out_ref, pooled):
    i = pl.program_id(0)
    a = a_ref[...]
    s = (a[0] + a[1] + hs_ref[...]) * disb_ref[...] + b3_ref[...]
    h = jnp.maximum(s, 0.0)
    bt = bt_ref[...]
    contrib = jnp.zeros((NG, 16), jnp.float32)
    for k in range(8):
        oh = (lax.broadcasted_iota(jnp.int32, (NG, PBLK), 0)
              == bt[k][None, :]).astype(jnp.float32)
        contrib += jnp.dot(oh, h[:, 16 * k:16 * (k + 1)],
                           preferred_element_type=jnp.float32)

    @pl.when(i == 0)
    def _():
        pooled[...] = jnp.zeros_like(pooled)

    pooled[...] += contrib

    @pl.when(i == GRID - 1)
    def _():
        emb = jnp.dot(pooled[...], we_ref[...],
                      preferred_element_type=jnp.float32) + be_ref[...][None, :]
        emb_ref[...] = emb
        out_ref[...] = (jnp.dot(jnp.maximum(emb, 0.0), wc_ref[...],
                                preferred_element_type=jnp.float32)
                        + bc_ref[...][None, :])


def _pk_spec(width=128):
    return pl.BlockSpec((PBLK, width), lambda i: (i, 0))


def _acc_spec():
    return pl.BlockSpec((2, PBLK, 128), lambda i: (0, i, 0))


def _full_spec(shape):
    nd = len(shape)
    return pl.BlockSpec(shape, lambda i: (0,) * nd)


def kernel(x, edge_index, batch, W1, b1, W2, b2, W3, b3, We, be, Wc, bc):
    _sc_edge_pass1, _sc_edge_pass2, _sc_deg_pass = _sc_kernels()
    ei_p = edge_index
    x_p = jnp.pad(x.astype(jnp.bfloat16),
                  ((0, NP - N), (0, 0))).reshape(NP // 8, 1024)
    batch_p = jnp.pad(batch, (0, NP - N), constant_values=NG)
    batch_t = batch_p.reshape(NP // 8, 8).T
    eye8 = jnp.eye(8, dtype=jnp.float32)
    w1_big = jnp.concatenate([jnp.kron(eye8, W1[:, :16]),
                              jnp.kron(eye8, W1[:, 16:])],
                             axis=1).astype(jnp.bfloat16)
    w2a = jnp.kron(eye8, W2[:16, :])
    w2b = jnp.kron(eye8, W2[16:, :])
    w3_bd = jnp.kron(eye8, W3)
    b1a = jnp.tile(b1[:16], 8)
    b1b = jnp.tile(b1[16:], 8)
    b2t = jnp.tile(b2, 8)
    b3t = jnp.tile(b3, 8)

    def pk(a):
        return a.reshape(a.shape[:-2] + (NP // 8, 128))

    def rows(a):
        return a.reshape(NP, 16)

    dacc = pk(_sc_deg_pass(ei_p))

    p1 = pl.pallas_call(
        _tc1a_body,
        grid=(GRID,),
        in_specs=[pl.BlockSpec((PBLK, 1024), lambda i: (i, 0)),
                  _full_spec((1024, 256))],
        out_specs=_pk_spec(256),
        out_shape=jax.ShapeDtypeStruct((NP // 8, 256), jnp.float32),
    )(x_p, w1_big)

    disb, hs1a, hs1b = pl.pallas_call(
        _tc1b_body,
        grid=(GRID,),
        in_specs=[_pk_spec(256), _acc_spec()],
        out_specs=[_pk_spec(), _pk_spec(), _pk_spec()],
        out_shape=[jax.ShapeDtypeStruct((NP // 8, 128), jnp.float32)] * 3,
    )(p1, dacc)

    acc1a, acc1b = _sc_edge_pass2(ei_p, rows(hs1a), rows(hs1b))

    hs2 = pl.pallas_call(
        _tc2_body,
        grid=(GRID,),
        in_specs=[_acc_spec(), _acc_spec(), _pk_spec(), _pk_spec(),
                  _pk_spec(), _full_spec((128, 128)), _full_spec((128, 128)),
                  _full_spec((128,)), _full_spec((128,))],
        out_specs=_pk_spec(),
        out_shape=jax.ShapeDtypeStruct((NP // 8, 128), jnp.float32),
    )(pk(acc1a), pk(acc1b), hs1a, hs1b, disb, w2a, w2b, b1a, b1b)

    acc2 = _sc_edge_pass1(ei_p, rows(hs2))

    hs3 = pl.pallas_call(
        _tc3_body,
        grid=(GRID,),
        in_specs=[_acc_spec(), _pk_spec(), _pk_spec(),
                  _full_spec((128, 128)), _full_spec((128,))],
        out_specs=_pk_spec(),
        out_shape=jax.ShapeDtypeStruct((NP // 8, 128), jnp.float32),
    )(pk(acc2), hs2, disb, w3_bd, b2t)

    acc3 = _sc_edge_pass1(ei_p, rows(hs3))

    embedding, output = pl.pallas_call(
        _tc4_body,
        grid=(GRID,),
        in_specs=[_acc_spec(), _pk_spec(), _pk_spec(),
                  _full_spec((128,)), pl.BlockSpec((8, PBLK), lambda i: (0, i)),
                  _full_spec((16, 16)), _full_spec((16,)),
                  _full_spec((16, 10)), _full_spec((10,))],
        out_specs=[_full_spec((NG, 16)), _full_spec((NG, 10))],
        out_shape=[jax.ShapeDtypeStruct((NG, 16), jnp.float32),
                   jax.ShapeDtypeStruct((NG, 10), jnp.float32)],
        scratch_shapes=[pltpu.VMEM((NG, 16), jnp.float32)],
    )(pk(acc3), hs3, disb, b3t, batch_t, We, be, Wc, bc)

    return (embedding, output)
